# baseline (speedup 1.0000x reference)
import jax
import jax.numpy as jnp
from jax.experimental import pallas as pl
from jax.experimental.pallas import tpu as pltpu

N = 10000
BM = 400
NB = N // BM
CW = 3328
NCH = 3
NTAIL = N - CW * NCH
NZ = 48
S_F = NB + NZ
GRID = S_F + 1


def _xw_kernel(x_ref, w1_ref, o_ref):
    o_ref[...] = jnp.dot(x_ref[...], w1_ref[...],
                         preferred_element_type=jnp.float32)


def _zchunk(sp):
    g = (sp >= 24).astype(jnp.int32) + (sp >= 40).astype(jnp.int32)
    base = jnp.where(g == 0, 0, jnp.where(g == 1, 24, 40))
    n = 3 - g
    local = sp - base
    return 8 * g + local // n, g + local % n


def _embed_kernel(adjA_ref, adjB_ref, xw_ref, b1_ref, w2_ref, b2_ref,
                  tail_ref, znorm_ref, hw_ref, zacc_ref):
    s = pl.program_id(0)

    @pl.when(s == 0)
    def _init():
        hw_ref[...] = jnp.zeros(hw_ref.shape, hw_ref.dtype)

    @pl.when(s < NB)
    def _h_phase():
        i = s
        acc = jnp.dot(adjA_ref[...], xw_ref[...],
                      preferred_element_type=jnp.float32)
        h = jnp.maximum(acc + b1_ref[...], 0.0)
        hw_ref[pl.ds(i * BM, BM), :] = jnp.dot(
            h, w2_ref[...], preferred_element_type=jnp.float32)
        bcols = (i + 1) * BM // CW * CW
        for c in range(NCH):
            @pl.when(bcols <= c * CW)
            def _zero(c=c):
                adjA_ref[:, c * CW:(c + 1) * CW] = jnp.zeros(
                    (BM, CW), jnp.float32)
        adjA_ref[:, NCH * CW:] = jnp.zeros((BM, NTAIL), jnp.float32)
        zacc_ref[pl.ds(i * BM, BM), :] = jnp.dot(
            adjA_ref[...], hw_ref[...], preferred_element_type=jnp.float32)

    @pl.when((s >= NB) & (s < S_F))
    def _z_phase():
        i, c = _zchunk(s - NB)
        part = jnp.dot(adjB_ref[...], hw_ref[pl.ds(c * CW, CW), :],
                       preferred_element_type=jnp.float32)
        zacc_ref[pl.ds(i * BM, BM), :] = (
            zacc_ref[pl.ds(i * BM, BM), :] + part)

    @pl.when(s == S_F)
    def _f_phase():
        ht = hw_ref[pl.ds(CW * NCH, NTAIL), :].astype(jnp.bfloat16)
        tail = jnp.dot(tail_ref[...], ht,
                       preferred_element_type=jnp.float32)
        g = zacc_ref[...] + tail + b2_ref[...]
        nrm = jnp.sqrt(jnp.sum(g * g, axis=1, keepdims=True))
        znorm_ref[...] = (g / (nrm + 1e-12)).astype(jnp.bfloat16)


def _recon_kernel(z_ref, zall_ref, o_ref):
    prod = jax.lax.dot_general(
        z_ref[...], zall_ref[...],
        dimension_numbers=(((1,), (1,)), ((), ())),
        preferred_element_type=jnp.float32)
    o_ref[...] = jax.nn.sigmoid(prod)


def _adjA_index(s):
    return (jnp.minimum(s, NB - 1), 0)


def _adjB_index(s):
    sp = jnp.clip(s - NB, 0, NZ - 1)
    i, c = _zchunk(sp)
    return (i, c)


def kernel(x, adj, W1, b1, W2, b2):
    b1 = b1.reshape(1, -1)
    b2 = b2.reshape(1, -1)
    nfeat = W1.shape[0]
    nhid = W1.shape[1]
    ndim = W2.shape[1]

    xw = pl.pallas_call(
        _xw_kernel,
        out_shape=jax.ShapeDtypeStruct((N, nhid), jnp.float32),
    )(x, W1)

    adj_tail = adj[:, CW * NCH:].astype(jnp.bfloat16)

    znorm = pl.pallas_call(
        _embed_kernel,
        grid=(GRID,),
        in_specs=[
            pl.BlockSpec((BM, N), _adjA_index),
            pl.BlockSpec((BM, CW), _adjB_index),
            pl.BlockSpec((N, nhid), lambda s: (0, 0)),
            pl.BlockSpec((1, nhid), lambda s: (0, 0)),
            pl.BlockSpec((nhid, ndim), lambda s: (0, 0)),
            pl.BlockSpec((1, ndim), lambda s: (0, 0)),
            pl.BlockSpec((N, NTAIL), lambda s: (0, 0)),
        ],
        out_specs=pl.BlockSpec((N, ndim), lambda s: (0, 0)),
        out_shape=jax.ShapeDtypeStruct((N, ndim), jnp.bfloat16),
        scratch_shapes=[
            pltpu.VMEM((N, ndim), jnp.float32),
            pltpu.VMEM((N, ndim), jnp.float32),
        ],
        compiler_params=pltpu.CompilerParams(
            dimension_semantics=("arbitrary",),
            vmem_limit_bytes=100 * 1024 * 1024,
        ),
    )(adj, adj, xw, b1, W2, b2, adj_tail)

    recon = pl.pallas_call(
        _recon_kernel,
        grid=(NB,),
        in_specs=[
            pl.BlockSpec((BM, ndim), lambda i: (i, 0)),
            pl.BlockSpec((N, ndim), lambda i: (0, 0)),
        ],
        out_specs=pl.BlockSpec((BM, N), lambda i: (i, 0)),
        out_shape=jax.ShapeDtypeStruct((N, N), jnp.float32),
    )(znorm, znorm)

    return recon
